# baseline (speedup 1.0000x reference)
import functools

import jax
import jax.numpy as jnp
from jax import lax
from jax.experimental import pallas as pl
from jax.experimental.pallas import tpu as pltpu
from jax.experimental.pallas import tpu_sc as plsc

MAXK = 5
NLEV = 3
L = 16
NW = 32
NG = 8

N, R, H = 16, 1024, 4096
CH = 16
QR = 256
NQ = R // QR
NCH = QR // CH
BUFCAP = 32
UNITS = N * (H // 128)
UPW = UNITS // NW
QPW = UPW * NQ


def _insert_val(t, v):
    c = [v > t[j] for j in range(MAXK)]
    nt = list(t)
    for j in range(MAXK - 1, 0, -1):
        nt[j] = jnp.where(c[j], jnp.where(c[j - 1], t[j - 1], v), t[j])
    nt[0] = jnp.where(c[0], v, t[0])
    return nt


def _insert(t, i, v, rv):
    c = [v > t[j] for j in range(MAXK)]
    nt = list(t)
    ni = list(i)
    for j in range(MAXK - 1, 0, -1):
        nt[j] = jnp.where(c[j], jnp.where(c[j - 1], t[j - 1], v), t[j])
        ni[j] = jnp.where(c[j], jnp.where(c[j - 1], i[j - 1], rv), i[j])
    nt[0] = jnp.where(c[0], v, t[0])
    ni[0] = jnp.where(c[0], rv, i[0])
    return nt, ni


def _sc_body(r_hbm, klog, gpad, tauv,
             ksel_o, v0_o, i0_o, v1_o, i1_o, v2_o, i2_o,
             kg0_v, kg1_v, tau_v, tile_v, cl_v, rt_v, ri_v, thr_v, cnt_v,
             stv0_v, sti0_v, stv1_v, sti1_v, stv2_v, sti2_v, ksel_v,
             sem0, sem1):
    wid = lax.axis_index("s") * 2 + lax.axis_index("c")
    iota = lax.iota(jnp.int32, L)
    neginf = jnp.full((L,), -jnp.inf, jnp.float32)
    zero_i = jnp.zeros((L,), jnp.int32)

    pltpu.sync_copy(klog, kg0_v)
    pltpu.sync_copy(gpad, kg1_v)
    pltpu.sync_copy(tauv, tau_v)
    tau = tau_v[pl.ds(0, L)]
    khard = []
    ksel = []
    for lvl in range(NLEV):
        x = (kg0_v[lvl, pl.ds(0, L)] + kg1_v[lvl, pl.ds(0, L)]) / tau
        m = jnp.max(x)
        e = jnp.exp(x - m)
        p = e / jnp.sum(e)
        pm = jnp.max(p)
        am = jnp.min(jnp.where(p == pm, iota, L))
        kh = am + 1
        ksoft = jnp.sum(p * (iota + 1).astype(jnp.float32))
        khard.append(kh)
        ksel.append((kh.astype(jnp.float32) + ksoft) - ksoft)
    kv = jnp.zeros((L,), jnp.float32)
    for lvl in range(NLEV):
        kv = jnp.where(iota == lvl, ksel[lvl], kv)
    ksel_v[pl.ds(0, L)] = kv

    @pl.when(wid == 0)
    def _():
        pltpu.sync_copy(ksel_v, ksel_o)

    keeps = [[jnp.full((L,), khard[lvl], jnp.int32) > p for p in range(MAXK)]
             for lvl in range(NLEV)]

    n = wid // 2
    hw = (wid % 2) * UPW
    sems = (sem0, sem1)

    def _dma(qi, b):
        u = qi // NQ
        q = qi % NQ
        h0 = (hw + u) * 128
        return pltpu.make_async_copy(
            r_hbm.at[n, pl.ds(q * QR, QR), pl.ds(h0, 128)],
            tile_v.at[b], sems[b])

    _dma(0, 0).start()

    def quarter_body(qi, b, tile):
        @pl.when(qi + 1 < QPW)
        def _():
            _dma(qi + 1, 1 - b).start()
        _dma(qi, b).wait()
        u = qi // NQ
        q = qi % NQ
        h0 = (hw + u) * 128

        def group_body(g, _):
            col = g * L + iota
            gs = pl.ds(pl.multiple_of(g * L, L), L)
            rbase = jnp.full((L,), q * QR, jnp.int32)

            @pl.when(q == 0)
            def _():
                for p in range(MAXK):
                    rt_v[p, gs] = neginf
                    ri_v[p, gs] = zero_i
                seed = tile[0, gs]
                for r in range(1, MAXK):
                    seed = jnp.minimum(seed, tile[r, gs])
                thr_v[gs] = seed
                cnt_v[gs] = zero_i

            def flush():
                cnt = cnt_v[gs]
                t = tuple(rt_v[p, gs] for p in range(MAXK))
                i = tuple(ri_v[p, gs] for p in range(MAXK))
                maxr = jnp.max(cnt)

                def fb(it, ti):
                    tt, ii = list(ti[0]), list(ti[1])
                    vals = []
                    for k in range(4):
                        s = it * 4 + k
                        valid = s < cnt
                        ridx = plsc.load_gather(
                            cl_v, [jnp.full((L,), s, jnp.int32), col],
                            mask=valid)
                        ridx = jnp.where(valid, ridx, 0)
                        v = plsc.load_gather(tile, [ridx, col], mask=valid)
                        vals.append((jnp.where(valid, v, -jnp.inf), ridx))
                    for v, ridx in vals:
                        tt, ii = _insert(tt, ii, v, rbase + ridx)
                    return tuple(tt), tuple(ii)

                t, i = lax.fori_loop(0, (maxr + 3) // 4, fb, (t, i))
                for p in range(MAXK):
                    rt_v[p, gs] = t[p]
                    ri_v[p, gs] = i[p]
                thr_v[gs] = t[MAXK - 1]
                cnt_v[gs] = zero_i

            def chunk_body(c, _):
                thr = thr_v[gs]
                cnt = cnt_v[gs]
                vs = [tile[c * CH + j, gs] for j in range(CH)]
                ms = [v >= thr for v in vs]
                anyv = ms[0]
                for j in range(1, CH):
                    anyv = jnp.logical_or(anyv, ms[j])

                @pl.when(jnp.any(anyv))
                def _():
                    incs = [jnp.where(m, 1, 0) for m in ms]
                    poss = [cnt]
                    for j in range(1, CH):
                        poss.append(poss[j - 1] + incs[j - 1])
                    for j in range(CH):
                        plsc.store_scatter(
                            cl_v, [poss[j], col],
                            jnp.full((L,), c * CH + j, jnp.int32),
                            mask=ms[j])
                    newcnt = poss[CH - 1] + incs[CH - 1]
                    cnt_v[gs] = newcnt

                    @pl.when(jnp.max(newcnt) > BUFCAP - CH)
                    def _():
                        flush()
                return 0

            lax.fori_loop(0, NCH, chunk_body, 0)
            flush()

            @pl.when(q == NQ - 1)
            def _():
                t = [rt_v[p, gs] for p in range(MAXK)]
                i = [ri_v[p, gs] for p in range(MAXK)]
                for lvl, (sv, si) in enumerate(((stv0_v, sti0_v),
                                                (stv1_v, sti1_v),
                                                (stv2_v, sti2_v))):
                    for p in range(MAXK):
                        sv[p, pl.ds(pl.multiple_of(g * L, L), L)] = (
                            jnp.where(keeps[lvl][p], t[p], neginf))
                        si[p, pl.ds(pl.multiple_of(g * L, L), L)] = (
                            jnp.where(keeps[lvl][p], i[p], zero_i))
            return 0

        lax.fori_loop(0, NG, group_body, 0)

        @pl.when(q == NQ - 1)
        def _():
            for sv, si, vo, io in ((stv0_v, sti0_v, v0_o, i0_o),
                                   (stv1_v, sti1_v, v1_o, i1_o),
                                   (stv2_v, sti2_v, v2_o, i2_o)):
                pltpu.sync_copy(sv, vo.at[n, :, pl.ds(h0, 128)])
                pltpu.sync_copy(si, io.at[n, :, pl.ds(h0, 128)])
        return 0

    def pair_body(q2, _):
        quarter_body(q2 * 2, 0, tile_v.at[0])
        quarter_body(q2 * 2 + 1, 1, tile_v.at[1])
        return 0

    lax.fori_loop(0, QPW // 2, pair_body, 0)


@jax.jit
def _run(r_combined, klog_pad, g_pad, tau_vec):
    big_f = jax.ShapeDtypeStruct((N, MAXK, H), jnp.float32)
    big_i = jax.ShapeDtypeStruct((N, MAXK, H), jnp.int32)
    out_type = (jax.ShapeDtypeStruct((128,), jnp.float32),
                big_f, big_i, big_f, big_i, big_f, big_i)
    mesh = plsc.VectorSubcoreMesh(core_axis_name="c", subcore_axis_name="s",
                                  num_cores=2, num_subcores=16)
    f = pl.kernel(
        _sc_body,
        out_type=out_type,
        mesh=mesh,
        scratch_types=(
            [pltpu.VMEM((8, 128), jnp.float32),
             pltpu.VMEM((8, 128), jnp.float32),
             pltpu.VMEM((128,), jnp.float32),
             pltpu.VMEM((2, QR, 128), jnp.float32),
             pltpu.VMEM((BUFCAP, 128), jnp.int32),
             pltpu.VMEM((MAXK, 128), jnp.float32),
             pltpu.VMEM((MAXK, 128), jnp.int32),
             pltpu.VMEM((128,), jnp.float32),
             pltpu.VMEM((128,), jnp.int32)]
            + [pltpu.VMEM((MAXK, 128), dt)
               for _ in range(NLEV) for dt in (jnp.float32, jnp.int32)]
            + [pltpu.VMEM((128,), jnp.float32),
               pltpu.SemaphoreType.DMA,
               pltpu.SemaphoreType.DMA]
        ),
        compiler_params=pltpu.CompilerParams(needs_layout_passes=False),
    )
    return f(r_combined, klog_pad, g_pad, tau_vec)


def kernel(R_combined, k_logits, temperature):
    u = jax.random.uniform(jax.random.key(42), (NLEV, MAXK), jnp.float32,
                           1e-10, 1.0)
    g = -jnp.log(-jnp.log(u + 1e-10))
    klog_pad = jnp.full((8, 128), -1e30, jnp.float32).at[:NLEV, :MAXK].set(
        k_logits)
    g_pad = jnp.zeros((8, 128), jnp.float32).at[:NLEV, :MAXK].set(g)
    tau_vec = jnp.full((128,), temperature, jnp.float32)
    ksel128, v0, i0, v1, i1, v2, i2 = _run(R_combined, klog_pad, g_pad,
                                           tau_vec)
    return ksel128[:NLEV], [(v0, i0), (v1, i1), (v2, i2)]


# speedup vs baseline: 1.4754x; 1.4754x over previous
import functools

import jax
import jax.numpy as jnp
from jax import lax
from jax.experimental import pallas as pl
from jax.experimental.pallas import tpu as pltpu
from jax.experimental.pallas import tpu_sc as plsc

MAXK = 5
NLEV = 3
L = 16
NW = 32
NG = 8

N, R, H = 16, 1024, 4096
CH = 16
QR = 256
NQ = R // QR
NCH = QR // CH
BUFCAP = 32
UNITS = N * (H // 128)
UPW = UNITS // NW
QPW = UPW * NQ


def _insert_val(t, v):
    c = [v > t[j] for j in range(MAXK)]
    nt = list(t)
    for j in range(MAXK - 1, 0, -1):
        nt[j] = jnp.where(c[j], jnp.where(c[j - 1], t[j - 1], v), t[j])
    nt[0] = jnp.where(c[0], v, t[0])
    return nt


def _insert(t, i, v, rv):
    c = [v > t[j] for j in range(MAXK)]
    nt = list(t)
    ni = list(i)
    for j in range(MAXK - 1, 0, -1):
        nt[j] = jnp.where(c[j], jnp.where(c[j - 1], t[j - 1], v), t[j])
        ni[j] = jnp.where(c[j], jnp.where(c[j - 1], i[j - 1], rv), i[j])
    nt[0] = jnp.where(c[0], v, t[0])
    ni[0] = jnp.where(c[0], rv, i[0])
    return nt, ni


def _sc_body(r_hbm, klog, gpad, tauv,
             ksel_o, v0_o, i0_o, v1_o, i1_o, v2_o, i2_o,
             kg0_v, kg1_v, tau_v, tile_v, cl_v, rt_v, ri_v, thr_v, cnt_v,
             stv0_v, sti0_v, stv1_v, sti1_v, stv2_v, sti2_v, ksel_v,
             sem0, sem1):
    wid = lax.axis_index("s") * 2 + lax.axis_index("c")
    iota = lax.iota(jnp.int32, L)
    neginf = jnp.full((L,), -jnp.inf, jnp.float32)
    zero_i = jnp.zeros((L,), jnp.int32)

    pltpu.sync_copy(klog, kg0_v)
    pltpu.sync_copy(gpad, kg1_v)
    pltpu.sync_copy(tauv, tau_v)
    tau = tau_v[pl.ds(0, L)]
    khard = []
    ksel = []
    for lvl in range(NLEV):
        x = (kg0_v[lvl, pl.ds(0, L)] + kg1_v[lvl, pl.ds(0, L)]) / tau
        m = jnp.max(x)
        e = jnp.exp(x - m)
        p = e / jnp.sum(e)
        pm = jnp.max(p)
        am = jnp.min(jnp.where(p == pm, iota, L))
        kh = am + 1
        ksoft = jnp.sum(p * (iota + 1).astype(jnp.float32))
        khard.append(kh)
        ksel.append((kh.astype(jnp.float32) + ksoft) - ksoft)
    kv = jnp.zeros((L,), jnp.float32)
    for lvl in range(NLEV):
        kv = jnp.where(iota == lvl, ksel[lvl], kv)
    ksel_v[pl.ds(0, L)] = kv

    @pl.when(wid == 0)
    def _():
        pltpu.sync_copy(ksel_v, ksel_o)

    keeps = [[jnp.full((L,), khard[lvl], jnp.int32) > p for p in range(MAXK)]
             for lvl in range(NLEV)]

    n = wid // 2
    hw = (wid % 2) * UPW
    sems = (sem0, sem1)

    def _dma(qi, b):
        u = qi // NQ
        q = qi % NQ
        h0 = (hw + u) * 128
        return pltpu.make_async_copy(
            r_hbm.at[n, pl.ds(q * QR, QR), pl.ds(h0, 128)],
            tile_v.at[b], sems[b])

    _dma(0, 0).start()

    def quarter_body(qi, b, tile):
        @pl.when(qi + 1 < QPW)
        def _():
            _dma(qi + 1, 1 - b).start()
        _dma(qi, b).wait()
        u = qi // NQ
        q = qi % NQ
        h0 = (hw + u) * 128

        def group_body(g, _):
            col = g * L + iota
            gs = pl.ds(pl.multiple_of(g * L, L), L)
            rbase = jnp.full((L,), q * QR, jnp.int32)

            @pl.when(q == 0)
            def _():
                for p in range(MAXK):
                    rt_v[p, gs] = neginf
                    ri_v[p, gs] = zero_i
                seed = tile[0, gs]
                for r in range(1, MAXK):
                    seed = jnp.minimum(seed, tile[r, gs])
                thr_v[gs] = seed
                cnt_v[gs] = zero_i

            def flush():
                cnt = cnt_v[gs]
                t = tuple(rt_v[p, gs] for p in range(MAXK))
                i = tuple(ri_v[p, gs] for p in range(MAXK))
                maxr = jnp.max(cnt)

                def fb(it, ti):
                    tt, ii = list(ti[0]), list(ti[1])
                    vals = []
                    for k in range(4):
                        s = it * 4 + k
                        valid = s < cnt
                        ridx = plsc.load_gather(
                            cl_v, [jnp.full((L,), s, jnp.int32), col],
                            mask=valid)
                        ridx = jnp.where(valid, ridx, 0)
                        v = plsc.load_gather(tile, [ridx, col], mask=valid)
                        vals.append((jnp.where(valid, v, -jnp.inf), ridx))
                    for v, ridx in vals:
                        tt, ii = _insert(tt, ii, v, rbase + ridx)
                    return tuple(tt), tuple(ii)

                t, i = lax.fori_loop(0, (maxr + 3) // 4, fb, (t, i))
                for p in range(MAXK):
                    rt_v[p, gs] = t[p]
                    ri_v[p, gs] = i[p]
                thr_v[gs] = t[MAXK - 1]
                cnt_v[gs] = zero_i

            def chunk_body(c, _):
                thr = thr_v[gs]
                cnt = cnt_v[gs]
                vs = [tile[c * CH + j, gs] for j in range(CH)]
                ms = [v >= thr for v in vs]
                incs = [jnp.where(m, 1, 0) for m in ms]
                poss = [cnt]
                for j in range(1, CH):
                    poss.append(poss[j - 1] + incs[j - 1])
                for j in range(CH):
                    plsc.store_scatter(cl_v, [poss[j], col],
                                       jnp.full((L,), c * CH + j, jnp.int32),
                                       mask=ms[j])
                newcnt = poss[CH - 1] + incs[CH - 1]
                cnt_v[gs] = newcnt

                @pl.when(jnp.max(newcnt) > BUFCAP - CH)
                def _():
                    flush()
                return 0

            lax.fori_loop(0, NCH, chunk_body, 0)
            flush()

            @pl.when(q == NQ - 1)
            def _():
                t = [rt_v[p, gs] for p in range(MAXK)]
                i = [ri_v[p, gs] for p in range(MAXK)]
                for lvl, (sv, si) in enumerate(((stv0_v, sti0_v),
                                                (stv1_v, sti1_v),
                                                (stv2_v, sti2_v))):
                    for p in range(MAXK):
                        sv[p, pl.ds(pl.multiple_of(g * L, L), L)] = (
                            jnp.where(keeps[lvl][p], t[p], neginf))
                        si[p, pl.ds(pl.multiple_of(g * L, L), L)] = (
                            jnp.where(keeps[lvl][p], i[p], zero_i))
            return 0

        lax.fori_loop(0, NG, group_body, 0)

        @pl.when(q == NQ - 1)
        def _():
            for sv, si, vo, io in ((stv0_v, sti0_v, v0_o, i0_o),
                                   (stv1_v, sti1_v, v1_o, i1_o),
                                   (stv2_v, sti2_v, v2_o, i2_o)):
                pltpu.sync_copy(sv, vo.at[n, :, pl.ds(h0, 128)])
                pltpu.sync_copy(si, io.at[n, :, pl.ds(h0, 128)])
        return 0

    def pair_body(q2, _):
        quarter_body(q2 * 2, 0, tile_v.at[0])
        quarter_body(q2 * 2 + 1, 1, tile_v.at[1])
        return 0

    lax.fori_loop(0, QPW // 2, pair_body, 0)


@jax.jit
def _run(r_combined, klog_pad, g_pad, tau_vec):
    big_f = jax.ShapeDtypeStruct((N, MAXK, H), jnp.float32)
    big_i = jax.ShapeDtypeStruct((N, MAXK, H), jnp.int32)
    out_type = (jax.ShapeDtypeStruct((128,), jnp.float32),
                big_f, big_i, big_f, big_i, big_f, big_i)
    mesh = plsc.VectorSubcoreMesh(core_axis_name="c", subcore_axis_name="s",
                                  num_cores=2, num_subcores=16)
    f = pl.kernel(
        _sc_body,
        out_type=out_type,
        mesh=mesh,
        scratch_types=(
            [pltpu.VMEM((8, 128), jnp.float32),
             pltpu.VMEM((8, 128), jnp.float32),
             pltpu.VMEM((128,), jnp.float32),
             pltpu.VMEM((2, QR, 128), jnp.float32),
             pltpu.VMEM((BUFCAP, 128), jnp.int32),
             pltpu.VMEM((MAXK, 128), jnp.float32),
             pltpu.VMEM((MAXK, 128), jnp.int32),
             pltpu.VMEM((128,), jnp.float32),
             pltpu.VMEM((128,), jnp.int32)]
            + [pltpu.VMEM((MAXK, 128), dt)
               for _ in range(NLEV) for dt in (jnp.float32, jnp.int32)]
            + [pltpu.VMEM((128,), jnp.float32),
               pltpu.SemaphoreType.DMA,
               pltpu.SemaphoreType.DMA]
        ),
        compiler_params=pltpu.CompilerParams(needs_layout_passes=False),
    )
    return f(r_combined, klog_pad, g_pad, tau_vec)


def kernel(R_combined, k_logits, temperature):
    u = jax.random.uniform(jax.random.key(42), (NLEV, MAXK), jnp.float32,
                           1e-10, 1.0)
    g = -jnp.log(-jnp.log(u + 1e-10))
    klog_pad = jnp.full((8, 128), -1e30, jnp.float32).at[:NLEV, :MAXK].set(
        k_logits)
    g_pad = jnp.zeros((8, 128), jnp.float32).at[:NLEV, :MAXK].set(g)
    tau_vec = jnp.full((128,), temperature, jnp.float32)
    ksel128, v0, i0, v1, i1, v2, i2 = _run(R_combined, klog_pad, g_pad,
                                           tau_vec)
    return ksel128[:NLEV], [(v0, i0), (v1, i1), (v2, i2)]


# speedup vs baseline: 1.5926x; 1.0794x over previous
import functools

import jax
import jax.numpy as jnp
from jax import lax
from jax.experimental import pallas as pl
from jax.experimental.pallas import tpu as pltpu
from jax.experimental.pallas import tpu_sc as plsc

MAXK = 5
NLEV = 3
L = 16
NW = 32
NG = 8

N, R, H = 16, 1024, 4096
CH = 16
QR = 256
NQ = R // QR
NCH = QR // CH
BUFCAP = 48
UNITS = N * (H // 128)
UPW = UNITS // NW
QPW = UPW * NQ


def _insert_val(t, v):
    c = [v > t[j] for j in range(MAXK)]
    nt = list(t)
    for j in range(MAXK - 1, 0, -1):
        nt[j] = jnp.where(c[j], jnp.where(c[j - 1], t[j - 1], v), t[j])
    nt[0] = jnp.where(c[0], v, t[0])
    return nt


def _insert(t, i, v, rv):
    c = [v > t[j] for j in range(MAXK)]
    nt = list(t)
    ni = list(i)
    for j in range(MAXK - 1, 0, -1):
        nt[j] = jnp.where(c[j], jnp.where(c[j - 1], t[j - 1], v), t[j])
        ni[j] = jnp.where(c[j], jnp.where(c[j - 1], i[j - 1], rv), i[j])
    nt[0] = jnp.where(c[0], v, t[0])
    ni[0] = jnp.where(c[0], rv, i[0])
    return nt, ni


def _sc_body(r_hbm, klog, gpad, tauv,
             ksel_o, v0_o, i0_o, v1_o, i1_o, v2_o, i2_o,
             kg0_v, kg1_v, tau_v, tile_v, cl_v, rt_v, ri_v, thr_v, cnt_v,
             stv0_v, sti0_v, stv1_v, sti1_v, stv2_v, sti2_v, ksel_v,
             sem0, sem1):
    wid = lax.axis_index("s") * 2 + lax.axis_index("c")
    iota = lax.iota(jnp.int32, L)
    neginf = jnp.full((L,), -jnp.inf, jnp.float32)
    zero_i = jnp.zeros((L,), jnp.int32)

    pltpu.sync_copy(klog, kg0_v)
    pltpu.sync_copy(gpad, kg1_v)
    pltpu.sync_copy(tauv, tau_v)
    tau = tau_v[pl.ds(0, L)]
    khard = []
    ksel = []
    for lvl in range(NLEV):
        x = (kg0_v[lvl, pl.ds(0, L)] + kg1_v[lvl, pl.ds(0, L)]) / tau
        m = jnp.max(x)
        e = jnp.exp(x - m)
        p = e / jnp.sum(e)
        pm = jnp.max(p)
        am = jnp.min(jnp.where(p == pm, iota, L))
        kh = am + 1
        ksoft = jnp.sum(p * (iota + 1).astype(jnp.float32))
        khard.append(kh)
        ksel.append((kh.astype(jnp.float32) + ksoft) - ksoft)
    kv = jnp.zeros((L,), jnp.float32)
    for lvl in range(NLEV):
        kv = jnp.where(iota == lvl, ksel[lvl], kv)
    ksel_v[pl.ds(0, L)] = kv

    @pl.when(wid == 0)
    def _():
        pltpu.sync_copy(ksel_v, ksel_o)

    keeps = [[jnp.full((L,), khard[lvl], jnp.int32) > p for p in range(MAXK)]
             for lvl in range(NLEV)]

    n = wid // 2
    hw = (wid % 2) * UPW
    sems = (sem0, sem1)

    def _dma(qi, b):
        u = qi // NQ
        q = qi % NQ
        h0 = (hw + u) * 128
        return pltpu.make_async_copy(
            r_hbm.at[n, pl.ds(q * QR, QR), pl.ds(h0, 128)],
            tile_v.at[b], sems[b])

    _dma(0, 0).start()

    def quarter_body(qi, b, tile):
        @pl.when(qi + 1 < QPW)
        def _():
            _dma(qi + 1, 1 - b).start()
        _dma(qi, b).wait()
        u = qi // NQ
        q = qi % NQ
        h0 = (hw + u) * 128

        def group_body(g, _):
            col = g * L + iota
            gs = pl.ds(pl.multiple_of(g * L, L), L)
            rbase = jnp.full((L,), q * QR, jnp.int32)

            @pl.when(q == 0)
            def _():
                for p in range(MAXK):
                    rt_v[p, gs] = neginf
                    ri_v[p, gs] = zero_i
                seed = tile[0, gs]
                for r in range(1, MAXK):
                    seed = jnp.minimum(seed, tile[r, gs])
                thr_v[gs] = seed
                cnt_v[gs] = zero_i

            def flush():
                cnt = cnt_v[gs]
                t = tuple(rt_v[p, gs] for p in range(MAXK))
                i = tuple(ri_v[p, gs] for p in range(MAXK))
                maxr = jnp.max(cnt)

                def fb(it, ti):
                    tt, ii = list(ti[0]), list(ti[1])
                    vals = []
                    for k in range(4):
                        s = it * 4 + k
                        valid = s < cnt
                        ridx = plsc.load_gather(
                            cl_v, [jnp.full((L,), s, jnp.int32), col],
                            mask=valid)
                        ridx = jnp.where(valid, ridx, 0)
                        v = plsc.load_gather(tile, [ridx, col], mask=valid)
                        vals.append((jnp.where(valid, v, -jnp.inf), ridx))
                    for v, ridx in vals:
                        tt, ii = _insert(tt, ii, v, rbase + ridx)
                    return tuple(tt), tuple(ii)

                t, i = lax.fori_loop(0, (maxr + 3) // 4, fb, (t, i))
                for p in range(MAXK):
                    rt_v[p, gs] = t[p]
                    ri_v[p, gs] = i[p]
                thr_v[gs] = t[MAXK - 1]
                cnt_v[gs] = zero_i

            def chunk_body(c, _):
                thr = thr_v[gs]
                cnt = cnt_v[gs]
                vs = [tile[c * CH + j, gs] for j in range(CH)]
                ms = [v >= thr for v in vs]
                incs = [jnp.where(m, 1, 0) for m in ms]
                poss = [cnt]
                for j in range(1, CH):
                    poss.append(poss[j - 1] + incs[j - 1])
                for j in range(CH):
                    plsc.store_scatter(cl_v, [poss[j], col],
                                       jnp.full((L,), c * CH + j, jnp.int32),
                                       mask=ms[j])
                cnt_v[gs] = poss[CH - 1] + incs[CH - 1]

                @pl.when(jnp.max(cnt) > CH)
                def _():
                    flush()
                return 0

            lax.fori_loop(0, NCH, chunk_body, 0)
            flush()

            @pl.when(q == NQ - 1)
            def _():
                t = [rt_v[p, gs] for p in range(MAXK)]
                i = [ri_v[p, gs] for p in range(MAXK)]
                for lvl, (sv, si) in enumerate(((stv0_v, sti0_v),
                                                (stv1_v, sti1_v),
                                                (stv2_v, sti2_v))):
                    for p in range(MAXK):
                        sv[p, pl.ds(pl.multiple_of(g * L, L), L)] = (
                            jnp.where(keeps[lvl][p], t[p], neginf))
                        si[p, pl.ds(pl.multiple_of(g * L, L), L)] = (
                            jnp.where(keeps[lvl][p], i[p], zero_i))
            return 0

        lax.fori_loop(0, NG, group_body, 0)

        @pl.when(q == NQ - 1)
        def _():
            for sv, si, vo, io in ((stv0_v, sti0_v, v0_o, i0_o),
                                   (stv1_v, sti1_v, v1_o, i1_o),
                                   (stv2_v, sti2_v, v2_o, i2_o)):
                pltpu.sync_copy(sv, vo.at[n, :, pl.ds(h0, 128)])
                pltpu.sync_copy(si, io.at[n, :, pl.ds(h0, 128)])
        return 0

    def pair_body(q2, _):
        quarter_body(q2 * 2, 0, tile_v.at[0])
        quarter_body(q2 * 2 + 1, 1, tile_v.at[1])
        return 0

    lax.fori_loop(0, QPW // 2, pair_body, 0)


@jax.jit
def _run(r_combined, klog_pad, g_pad, tau_vec):
    big_f = jax.ShapeDtypeStruct((N, MAXK, H), jnp.float32)
    big_i = jax.ShapeDtypeStruct((N, MAXK, H), jnp.int32)
    out_type = (jax.ShapeDtypeStruct((128,), jnp.float32),
                big_f, big_i, big_f, big_i, big_f, big_i)
    mesh = plsc.VectorSubcoreMesh(core_axis_name="c", subcore_axis_name="s",
                                  num_cores=2, num_subcores=16)
    f = pl.kernel(
        _sc_body,
        out_type=out_type,
        mesh=mesh,
        scratch_types=(
            [pltpu.VMEM((8, 128), jnp.float32),
             pltpu.VMEM((8, 128), jnp.float32),
             pltpu.VMEM((128,), jnp.float32),
             pltpu.VMEM((2, QR, 128), jnp.float32),
             pltpu.VMEM((BUFCAP, 128), jnp.int32),
             pltpu.VMEM((MAXK, 128), jnp.float32),
             pltpu.VMEM((MAXK, 128), jnp.int32),
             pltpu.VMEM((128,), jnp.float32),
             pltpu.VMEM((128,), jnp.int32)]
            + [pltpu.VMEM((MAXK, 128), dt)
               for _ in range(NLEV) for dt in (jnp.float32, jnp.int32)]
            + [pltpu.VMEM((128,), jnp.float32),
               pltpu.SemaphoreType.DMA,
               pltpu.SemaphoreType.DMA]
        ),
        compiler_params=pltpu.CompilerParams(needs_layout_passes=False),
    )
    return f(r_combined, klog_pad, g_pad, tau_vec)


def kernel(R_combined, k_logits, temperature):
    u = jax.random.uniform(jax.random.key(42), (NLEV, MAXK), jnp.float32,
                           1e-10, 1.0)
    g = -jnp.log(-jnp.log(u + 1e-10))
    klog_pad = jnp.full((8, 128), -1e30, jnp.float32).at[:NLEV, :MAXK].set(
        k_logits)
    g_pad = jnp.zeros((8, 128), jnp.float32).at[:NLEV, :MAXK].set(g)
    tau_vec = jnp.full((128,), temperature, jnp.float32)
    ksel128, v0, i0, v1, i1, v2, i2 = _run(R_combined, klog_pad, g_pad,
                                           tau_vec)
    return ksel128[:NLEV], [(v0, i0), (v1, i1), (v2, i2)]


# speedup vs baseline: 1.5931x; 1.0003x over previous
import jax
import jax.numpy as jnp
from jax import lax
from jax.experimental import pallas as pl
from jax.experimental.pallas import tpu as pltpu
from jax.experimental.pallas import tpu_sc as plsc

MAXK = 5
NLEV = 3
L = 16
NW = 32
NG = 8

N, R, H = 16, 1024, 4096
CH = 16
QR = 256
NQ = R // QR
NCH = QR // CH
BUFCAP = 48
UNITS = N * (H // 128)
UPW = UNITS // NW
QPW = UPW * NQ


def _insert(t, i, v, rv):
    c = [v > t[j] for j in range(MAXK)]
    nt = list(t)
    ni = list(i)
    for j in range(MAXK - 1, 0, -1):
        nt[j] = jnp.where(c[j], jnp.where(c[j - 1], t[j - 1], v), t[j])
        ni[j] = jnp.where(c[j], jnp.where(c[j - 1], i[j - 1], rv), i[j])
    nt[0] = jnp.where(c[0], v, t[0])
    ni[0] = jnp.where(c[0], rv, i[0])
    return nt, ni


def _sc_body(r_hbm, klog, gpad, tauv,
             ksel_o, v0_o, i0_o, v1_o, i1_o, v2_o, i2_o,
             kg0_v, kg1_v, tau_v, tile_v, cl_v, rt_v, ri_v, thr_v, cnt_v,
             stv0_v, sti0_v, stv1_v, sti1_v, stv2_v, sti2_v, ksel_v,
             sem0, sem1):
    wid = lax.axis_index("s") * 2 + lax.axis_index("c")
    iota = lax.iota(jnp.int32, L)
    neginf = jnp.full((L,), -jnp.inf, jnp.float32)
    zero_i = jnp.zeros((L,), jnp.int32)

    pltpu.sync_copy(klog, kg0_v)
    pltpu.sync_copy(gpad, kg1_v)
    pltpu.sync_copy(tauv, tau_v)
    tau = tau_v[pl.ds(0, L)]
    khard = []
    ksel = []
    for lvl in range(NLEV):
        x = (kg0_v[lvl, pl.ds(0, L)] + kg1_v[lvl, pl.ds(0, L)]) / tau
        m = jnp.max(x)
        e = jnp.exp(x - m)
        p = e / jnp.sum(e)
        pm = jnp.max(p)
        am = jnp.min(jnp.where(p == pm, iota, L))
        kh = am + 1
        ksoft = jnp.sum(p * (iota + 1).astype(jnp.float32))
        khard.append(kh)
        ksel.append((kh.astype(jnp.float32) + ksoft) - ksoft)
    kv = jnp.zeros((L,), jnp.float32)
    for lvl in range(NLEV):
        kv = jnp.where(iota == lvl, ksel[lvl], kv)
    ksel_v[pl.ds(0, L)] = kv

    @pl.when(wid == 0)
    def _():
        pltpu.sync_copy(ksel_v, ksel_o)

    keeps = [[jnp.full((L,), khard[lvl], jnp.int32) > p for p in range(MAXK)]
             for lvl in range(NLEV)]

    n = wid // 2
    hw = (wid % 2) * UPW
    sems = (sem0, sem1)

    def _dma(qi, b):
        u = qi // NQ
        q = qi % NQ
        h0 = (hw + u) * 128
        return pltpu.make_async_copy(
            r_hbm.at[n, pl.ds(q * QR, QR), pl.ds(h0, 128)],
            tile_v.at[b], sems[b])

    _dma(0, 0).start()

    def quarter_body(qi, b, tile):
        @pl.when(qi + 1 < QPW)
        def _():
            _dma(qi + 1, 1 - b).start()
        _dma(qi, b).wait()
        u = qi // NQ
        q = qi % NQ
        h0 = (hw + u) * 128

        def group_body(g, _):
            col = g * L + iota
            gs = pl.ds(pl.multiple_of(g * L, L), L)
            rbase = jnp.full((L,), q * QR, jnp.int32)

            @pl.when(q == 0)
            def _():
                for p in range(MAXK):
                    rt_v[p, gs] = neginf
                    ri_v[p, gs] = zero_i
                seed = tile[0, gs]
                for r in range(1, MAXK):
                    seed = jnp.minimum(seed, tile[r, gs])
                thr_v[gs] = seed
                cnt_v[gs] = zero_i

            def flush():
                cnt = cnt_v[gs]
                t = tuple(rt_v[p, gs] for p in range(MAXK))
                i = tuple(ri_v[p, gs] for p in range(MAXK))
                maxr = jnp.max(cnt)

                def fb(it, ti):
                    tt, ii = list(ti[0]), list(ti[1])
                    vals = []
                    for k in range(4):
                        s = it * 4 + k
                        valid = s < cnt
                        ridx = plsc.load_gather(
                            cl_v, [jnp.full((L,), s, jnp.int32), col],
                            mask=valid)
                        ridx = jnp.where(valid, ridx, 0)
                        v = plsc.load_gather(tile, [ridx, col], mask=valid)
                        vals.append((jnp.where(valid, v, -jnp.inf), ridx))
                    for v, ridx in vals:
                        tt, ii = _insert(tt, ii, v, rbase + ridx)
                    return tuple(tt), tuple(ii)

                t, i = lax.fori_loop(0, (maxr + 3) // 4, fb, (t, i))
                for p in range(MAXK):
                    rt_v[p, gs] = t[p]
                    ri_v[p, gs] = i[p]
                thr_v[gs] = t[MAXK - 1]
                cnt_v[gs] = zero_i

            def chunk_body(c, _):
                thr = thr_v[gs]
                cnt = cnt_v[gs]
                vs = [tile[c * CH + j, gs] for j in range(CH)]
                ms = [v >= thr for v in vs]
                incs = [jnp.where(m, 1, 0) for m in ms]
                poss = [cnt]
                for j in range(1, CH):
                    poss.append(poss[j - 1] + incs[j - 1])
                for j in range(CH):
                    plsc.store_scatter(cl_v, [poss[j], col],
                                       jnp.full((L,), c * CH + j, jnp.int32),
                                       mask=ms[j])
                cnt_v[gs] = poss[CH - 1] + incs[CH - 1]

                @pl.when(jnp.max(cnt) > CH)
                def _():
                    flush()
                return 0

            lax.fori_loop(0, NCH, chunk_body, 0)
            flush()

            @pl.when(q == NQ - 1)
            def _():
                t = [rt_v[p, gs] for p in range(MAXK)]
                i = [ri_v[p, gs] for p in range(MAXK)]
                for lvl, (sv, si) in enumerate(((stv0_v, sti0_v),
                                                (stv1_v, sti1_v),
                                                (stv2_v, sti2_v))):
                    for p in range(MAXK):
                        sv[p, pl.ds(pl.multiple_of(g * L, L), L)] = (
                            jnp.where(keeps[lvl][p], t[p], neginf))
                        si[p, pl.ds(pl.multiple_of(g * L, L), L)] = (
                            jnp.where(keeps[lvl][p], i[p], zero_i))
            return 0

        lax.fori_loop(0, NG, group_body, 0)

        @pl.when(q == NQ - 1)
        def _():
            for sv, si, vo, io in ((stv0_v, sti0_v, v0_o, i0_o),
                                   (stv1_v, sti1_v, v1_o, i1_o),
                                   (stv2_v, sti2_v, v2_o, i2_o)):
                pltpu.sync_copy(sv, vo.at[n, :, pl.ds(h0, 128)])
                pltpu.sync_copy(si, io.at[n, :, pl.ds(h0, 128)])
        return 0

    def pair_body(q2, _):
        quarter_body(q2 * 2, 0, tile_v.at[0])
        quarter_body(q2 * 2 + 1, 1, tile_v.at[1])
        return 0

    lax.fori_loop(0, QPW // 2, pair_body, 0)


@jax.jit
def _run(r_combined, klog_pad, g_pad, tau_vec):
    big_f = jax.ShapeDtypeStruct((N, MAXK, H), jnp.float32)
    big_i = jax.ShapeDtypeStruct((N, MAXK, H), jnp.int32)
    out_type = (jax.ShapeDtypeStruct((128,), jnp.float32),
                big_f, big_i, big_f, big_i, big_f, big_i)
    mesh = plsc.VectorSubcoreMesh(core_axis_name="c", subcore_axis_name="s",
                                  num_cores=2, num_subcores=16)
    f = pl.kernel(
        _sc_body,
        out_type=out_type,
        mesh=mesh,
        scratch_types=(
            [pltpu.VMEM((8, 128), jnp.float32),
             pltpu.VMEM((8, 128), jnp.float32),
             pltpu.VMEM((128,), jnp.float32),
             pltpu.VMEM((2, QR, 128), jnp.float32),
             pltpu.VMEM((BUFCAP, 128), jnp.int32),
             pltpu.VMEM((MAXK, 128), jnp.float32),
             pltpu.VMEM((MAXK, 128), jnp.int32),
             pltpu.VMEM((128,), jnp.float32),
             pltpu.VMEM((128,), jnp.int32)]
            + [pltpu.VMEM((MAXK, 128), dt)
               for _ in range(NLEV) for dt in (jnp.float32, jnp.int32)]
            + [pltpu.VMEM((128,), jnp.float32),
               pltpu.SemaphoreType.DMA,
               pltpu.SemaphoreType.DMA]
        ),
        compiler_params=pltpu.CompilerParams(needs_layout_passes=False),
    )
    return f(r_combined, klog_pad, g_pad, tau_vec)


def kernel(R_combined, k_logits, temperature):
    u = jax.random.uniform(jax.random.key(42), (NLEV, MAXK), jnp.float32,
                           1e-10, 1.0)
    g = -jnp.log(-jnp.log(u + 1e-10))
    klog_pad = jnp.full((8, 128), -1e30, jnp.float32).at[:NLEV, :MAXK].set(
        k_logits)
    g_pad = jnp.zeros((8, 128), jnp.float32).at[:NLEV, :MAXK].set(g)
    tau_vec = jnp.full((128,), temperature, jnp.float32)
    ksel128, v0, i0, v1, i1, v2, i2 = _run(R_combined, klog_pad, g_pad,
                                           tau_vec)
    return ksel128[:NLEV], [(v0, i0), (v1, i1), (v2, i2)]
